# Initial kernel scaffold; baseline (speedup 1.0000x reference)
import jax
import jax.numpy as jnp
from jax.experimental import pallas as pl


def kernel(idxs, targs, table):
    raise NotImplementedError("write your pallas kernel here")


# speedup vs baseline: 1.6098x; 1.6098x over previous
import functools

import jax
import jax.numpy as jnp
from jax import lax
from jax.experimental import pallas as pl
from jax.experimental.pallas import tpu as pltpu
from jax.experimental.pallas import tpu_sc as plsc

V = 1000
D = 1000
N = 51200
NC, NS, L = 2, 16, 16
NW = NC * NS
PER_W = N // NW
CH = 32
NCH = PER_W // CH
LSE_PAD = 1024


def _lse_body(table_ref, out_ref):
    t = table_ref[...]
    m = jnp.max(t, axis=1)
    s = jnp.sum(jnp.exp(t - m[:, None]), axis=1)
    out_ref[...] = (m + jnp.log(s))[:, None]


def _compute_lse(table):
    return pl.pallas_call(
        _lse_body,
        out_shape=jax.ShapeDtypeStruct((V, 1), jnp.float32),
    )(table)


def _fin_body(part_ref, out_ref):
    out_ref[...] = jnp.full((1, 1), jnp.sum(part_ref[...]) / N, jnp.float32)


def _finalize(part):
    return pl.pallas_call(
        _fin_body,
        out_shape=jax.ShapeDtypeStruct((1, 1), jnp.float32),
    )(part)


def _sc_body(table_hbm, idx_hbm, targ_hbm, lse_hbm,
             out_hbm, part_hbm,
             idx_v, targ_v, lse_v, rows_v, acc_v, gsem):
    wid = lax.axis_index("s") * NC + lax.axis_index("c")
    base = wid * PER_W
    pltpu.sync_copy(idx_hbm.at[pl.ds(base, PER_W)], idx_v)
    pltpu.sync_copy(targ_hbm.at[pl.ds(base, PER_W)], targ_v)
    pltpu.sync_copy(lse_hbm, lse_v)

    def chunk(c, acc):
        off = c * CH
        pltpu.async_copy(
            table_hbm.at[idx_v.at[pl.ds(off, CH)]], rows_v, gsem
        ).wait()
        for g in range(CH // L):
            o = off + g * L
            ivec = idx_v[pl.ds(o, L)]
            tvec = targ_v[pl.ds(o, L)]
            rowi = lax.iota(jnp.int32, L) + g * L
            picked = plsc.load_gather(rows_v, [rowi, tvec])
            lseg = plsc.load_gather(lse_v, [ivec])
            acc = acc + (lseg - picked)
        pltpu.sync_copy(rows_v, out_hbm.at[pl.ds(base + off, CH)])
        return acc

    acc = lax.fori_loop(0, NCH, chunk, jnp.zeros((L,), jnp.float32))
    acc_v[...] = acc
    pltpu.sync_copy(acc_v, part_hbm.at[wid])


@functools.cache
def _sc_gather_fn():
    mesh = plsc.VectorSubcoreMesh(
        core_axis_name="c", subcore_axis_name="s",
        num_cores=NC, num_subcores=NS,
    )
    return pl.kernel(
        _sc_body,
        out_type=(
            jax.ShapeDtypeStruct((N, D), jnp.float32),
            jax.ShapeDtypeStruct((NW, L), jnp.float32),
        ),
        mesh=mesh,
        compiler_params=pltpu.CompilerParams(
            use_tc_tiling_on_sc=False, needs_layout_passes=False
        ),
        scratch_types=[
            pltpu.VMEM((PER_W,), jnp.int32),
            pltpu.VMEM((PER_W,), jnp.int32),
            pltpu.VMEM((LSE_PAD,), jnp.float32),
            pltpu.VMEM((CH, D), jnp.float32),
            pltpu.VMEM((L,), jnp.float32),
            pltpu.SemaphoreType.DMA,
        ],
    )


def kernel(idxs, targs, table):
    idx_flat = idxs.reshape(-1)
    targ_flat = targs.reshape(-1)
    lse = _compute_lse(table)
    lse_pad = jnp.pad(lse[:, 0], (0, LSE_PAD - V))
    lgits2, part = _sc_gather_fn()(table, idx_flat, targ_flat, lse_pad)
    loss = _finalize(part)[0, 0]
    return (lgits2, loss)


# speedup vs baseline: 1.7006x; 1.0564x over previous
import functools

import jax
import jax.numpy as jnp
from jax import lax
from jax.experimental import pallas as pl
from jax.experimental.pallas import tpu as pltpu
from jax.experimental.pallas import tpu_sc as plsc

V = 1000
D = 1000
N = 51200
NC, NS, L = 2, 16, 16
NW = NC * NS
PER_W = N // NW
CH = 32
NCH = PER_W // CH
NB = 2
LSE_PAD = 1024


def _lse_body(table_ref, out_ref):
    t = table_ref[...]
    m = jnp.max(t, axis=1)
    s = jnp.sum(jnp.exp(t - m[:, None]), axis=1)
    out_ref[...] = (m + jnp.log(s))[:, None]


def _compute_lse(table):
    return pl.pallas_call(
        _lse_body,
        out_shape=jax.ShapeDtypeStruct((V, 1), jnp.float32),
    )(table)


def _fin_body(part_ref, out_ref):
    out_ref[...] = jnp.full((1, 1), jnp.sum(part_ref[...]) / N, jnp.float32)


def _finalize(part):
    return pl.pallas_call(
        _fin_body,
        out_shape=jax.ShapeDtypeStruct((1, 1), jnp.float32),
    )(part)


def _sc_body(table_hbm, idx_hbm, targ_hbm, lse_hbm,
             out_hbm, part_hbm,
             idx_v, targ_v, lse_v, rows_v, acc_v,
             gsem0, gsem1, ssem0, ssem1):
    gsems = (gsem0, gsem1)
    ssems = (ssem0, ssem1)
    wid = lax.axis_index("s") * NC + lax.axis_index("c")
    base = wid * PER_W
    pltpu.sync_copy(idx_hbm.at[pl.ds(base, PER_W)], idx_v)
    pltpu.sync_copy(targ_hbm.at[pl.ds(base, PER_W)], targ_v)
    pltpu.sync_copy(lse_hbm, lse_v)

    for b in range(NB):
        pltpu.async_copy(
            table_hbm.at[idx_v.at[pl.ds(b * CH, CH)]], rows_v.at[b], gsems[b]
        )

    def outer(o, acc):
        for b in range(NB):
            c = o * NB + b
            off = c * CH
            buf = rows_v.at[b]
            pltpu.make_async_copy(
                table_hbm.at[idx_v.at[pl.ds(off, CH)]], buf, gsems[b]
            ).wait()
            sc_desc = pltpu.async_copy(
                buf, out_hbm.at[pl.ds(base + off, CH)], ssems[b]
            )
            for g in range(CH // L):
                p = off + g * L
                ivec = idx_v[pl.ds(p, L)]
                tvec = targ_v[pl.ds(p, L)]
                rowi = lax.iota(jnp.int32, L) + g * L
                picked = plsc.load_gather(buf, [rowi, tvec])
                lseg = plsc.load_gather(lse_v, [ivec])
                acc = acc + (lseg - picked)
            sc_desc.wait()
            nc = c + NB

            @pl.when(nc < NCH)
            def _():
                pltpu.async_copy(
                    table_hbm.at[idx_v.at[pl.ds(nc * CH, CH)]], buf, gsems[b]
                )
        return acc

    acc = lax.fori_loop(0, NCH // NB, outer, jnp.zeros((L,), jnp.float32))
    acc_v[...] = acc
    pltpu.sync_copy(acc_v, part_hbm.at[wid])


@functools.cache
def _sc_gather_fn():
    mesh = plsc.VectorSubcoreMesh(
        core_axis_name="c", subcore_axis_name="s",
        num_cores=NC, num_subcores=NS,
    )
    return pl.kernel(
        _sc_body,
        out_type=(
            jax.ShapeDtypeStruct((N, D), jnp.float32),
            jax.ShapeDtypeStruct((NW, L), jnp.float32),
        ),
        mesh=mesh,
        compiler_params=pltpu.CompilerParams(
            use_tc_tiling_on_sc=False, needs_layout_passes=False
        ),
        scratch_types=[
            pltpu.VMEM((PER_W,), jnp.int32),
            pltpu.VMEM((PER_W,), jnp.int32),
            pltpu.VMEM((LSE_PAD,), jnp.float32),
            pltpu.VMEM((NB, CH, D), jnp.float32),
            pltpu.VMEM((L,), jnp.float32),
            pltpu.SemaphoreType.DMA,
            pltpu.SemaphoreType.DMA,
            pltpu.SemaphoreType.DMA,
            pltpu.SemaphoreType.DMA,
        ],
    )


def kernel(idxs, targs, table):
    idx_flat = idxs.reshape(-1)
    targ_flat = targs.reshape(-1)
    lse = _compute_lse(table)
    lse_pad = jnp.pad(lse[:, 0], (0, LSE_PAD - V))
    lgits2, part = _sc_gather_fn()(table, idx_flat, targ_flat, lse_pad)
    loss = _finalize(part)[0, 0]
    return (lgits2, loss)


# speedup vs baseline: 2.5631x; 1.5072x over previous
import functools

import jax
import jax.numpy as jnp
from jax import lax
from jax.experimental import pallas as pl
from jax.experimental.pallas import tpu as pltpu
from jax.experimental.pallas import tpu_sc as plsc

V = 1000
D = 1000
DP = 1024
N = 51200
NC, NS, L = 2, 16, 16
NW = NC * NS
PER_W = N // NW
CH = 32
NCH = PER_W // CH
NB = 2
LSE_PAD = 1024
TFLAT = V * D + 8


def _lse_body(table_ref, out_ref):
    t = table_ref[...]
    m = jnp.max(t, axis=1)
    s = jnp.sum(jnp.exp(t - m[:, None]), axis=1)
    out_ref[...] = (m + jnp.log(s))[:, None]


def _compute_lse(table):
    return pl.pallas_call(
        _lse_body,
        out_shape=jax.ShapeDtypeStruct((V, 1), jnp.float32),
    )(table)


def _fin_body(part_ref, out_ref):
    out_ref[...] = jnp.full((1, 1), jnp.sum(part_ref[...]) / N, jnp.float32)


def _finalize(part):
    return pl.pallas_call(
        _fin_body,
        out_shape=jax.ShapeDtypeStruct((1, 1), jnp.float32),
    )(part)


def _gather_body(table_hbm, idx_hbm, out_hbm,
                 idx_v, rows_v, gsem0, gsem1, ssem0, ssem1):
    gsems = (gsem0, gsem1)
    ssems = (ssem0, ssem1)
    wid = lax.axis_index("s") * NC + lax.axis_index("c")
    base = wid * PER_W
    pltpu.sync_copy(idx_hbm.at[pl.ds(base, PER_W)], idx_v)

    for b in range(NB):
        pltpu.async_copy(
            table_hbm.at[idx_v.at[pl.ds(b * CH, CH)]], rows_v.at[b], gsems[b]
        )

    def outer(o, _):
        for b in range(NB):
            c = o * NB + b
            off = c * CH
            buf = rows_v.at[b]
            pltpu.make_async_copy(
                table_hbm.at[idx_v.at[pl.ds(off, CH)]], buf, gsems[b]
            ).wait()
            sc_desc = pltpu.async_copy(
                buf, out_hbm.at[pl.ds(base + off, CH)], ssems[b]
            )
            sc_desc.wait()
            nc = c + NB

            @pl.when(nc < NCH)
            def _():
                pltpu.async_copy(
                    table_hbm.at[idx_v.at[pl.ds(nc * CH, CH)]], buf, gsems[b]
                )
        return 0

    lax.fori_loop(0, NCH // NB, outer, 0)


@functools.cache
def _gather_fn():
    mesh = plsc.VectorSubcoreMesh(
        core_axis_name="c", subcore_axis_name="s",
        num_cores=NC, num_subcores=NS,
    )
    return pl.kernel(
        _gather_body,
        out_type=jax.ShapeDtypeStruct((N, DP), jnp.float32),
        mesh=mesh,
        compiler_params=pltpu.CompilerParams(use_tc_tiling_on_sc=True),
        scratch_types=[
            pltpu.VMEM((PER_W,), jnp.int32),
            pltpu.VMEM((NB, CH, DP), jnp.float32),
            pltpu.SemaphoreType.DMA,
            pltpu.SemaphoreType.DMA,
            pltpu.SemaphoreType.DMA,
            pltpu.SemaphoreType.DMA,
        ],
    )


def _loss_body(tflat_hbm, idx_hbm, fidx_hbm, lse_hbm, part_hbm,
               idx_v, fidx_v, picked_v, lseg_v, acc_v, sem):
    wid = lax.axis_index("s") * NC + lax.axis_index("c")
    base = wid * PER_W
    pltpu.sync_copy(idx_hbm.at[pl.ds(base, PER_W)], idx_v)
    pltpu.sync_copy(fidx_hbm.at[pl.ds(base, PER_W)], fidx_v)
    pltpu.async_copy(tflat_hbm.at[fidx_v], picked_v, sem).wait()
    pltpu.async_copy(lse_hbm.at[idx_v], lseg_v, sem).wait()

    def red(g, acc):
        o = g * L
        return acc + (lseg_v[pl.ds(o, L)] - picked_v[pl.ds(o, L)])

    acc_v[...] = lax.fori_loop(0, PER_W // L, red, jnp.zeros((L,), jnp.float32))
    pltpu.sync_copy(acc_v, part_hbm.at[wid])


@functools.cache
def _loss_fn():
    mesh = plsc.VectorSubcoreMesh(
        core_axis_name="c", subcore_axis_name="s",
        num_cores=NC, num_subcores=NS,
    )
    return pl.kernel(
        _loss_body,
        out_type=jax.ShapeDtypeStruct((NW, L), jnp.float32),
        mesh=mesh,
        compiler_params=pltpu.CompilerParams(
            use_tc_tiling_on_sc=False, needs_layout_passes=False
        ),
        scratch_types=[
            pltpu.VMEM((PER_W,), jnp.int32),
            pltpu.VMEM((PER_W,), jnp.int32),
            pltpu.VMEM((PER_W,), jnp.float32),
            pltpu.VMEM((PER_W,), jnp.float32),
            pltpu.VMEM((L,), jnp.float32),
            pltpu.SemaphoreType.DMA,
        ],
    )


def kernel(idxs, targs, table):
    idx_flat = idxs.reshape(-1)
    targ_flat = targs.reshape(-1)
    fidx = idx_flat * D + targ_flat
    table_pad = jnp.pad(table, ((0, 0), (0, DP - D)))
    tflat = jnp.concatenate(
        [table.reshape(-1), jnp.zeros((TFLAT - V * D,), jnp.float32)]
    )
    lse = _compute_lse(table)
    lse_pad = jnp.pad(lse[:, 0], (0, LSE_PAD - V))
    lgits2 = _gather_fn()(table_pad, idx_flat)[:, :D]
    part = _loss_fn()(tflat, idx_flat, fidx, lse_pad)
    loss = _finalize(part)[0, 0]
    return (lgits2, loss)
